# Initial kernel scaffold; baseline (speedup 1.0000x reference)
import jax
import jax.numpy as jnp
from jax.experimental import pallas as pl


def kernel(X, v):
    raise NotImplementedError("write your pallas kernel here")


# speedup vs baseline: 7.0846x; 7.0846x over previous
import functools

import jax
import jax.numpy as jnp
import numpy as np
from jax import lax
from jax.experimental import pallas as pl
from jax.experimental.pallas import tpu as pltpu
from jax.experimental.pallas import tpu_sc as plsc

_FIELDS = 26
_EMBED = 16
_NPAIRS = (_FIELDS * (_FIELDS - 1)) // 2
_IU_R, _IU_C = np.triu_indices(_FIELDS, k=1)

_NC = 2
_NS = 16
_NW = _NC * _NS
_G = 4

_ROW = _FIELDS * _EMBED
_OROW = _NPAIRS * _EMBED


def _pair_products(rows_ref, out_ref, gb):
    rbase = gb * _FIELDS
    obase = gb * _OROW
    for p in range(_NPAIRS):
        i = int(_IU_R[p])
        j = int(_IU_C[p])
        a = rows_ref[rbase + i, pl.ds(j * _EMBED, _EMBED)]
        b = rows_ref[rbase + j, pl.ds(i * _EMBED, _EMBED)]
        out_ref[pl.ds(obase + p * _EMBED, _EMBED)] = a * b


def _sc_body(nb, ng, x_hbm, v_hbm, out_hbm,
             idx0, idx1, rows0, rows1, outv0, outv1,
             gsem0, gsem1, osem0, osem1):
    idx = (idx0, idx1)
    rows = (rows0, rows1)
    outv = (outv0, outv1)
    gsem = (gsem0, gsem1)
    osem = (osem0, osem1)

    wid = lax.axis_index("s") * _NC + lax.axis_index("c")
    base = wid * nb

    def start_gather(g, buf):
        pltpu.sync_copy(x_hbm.at[pl.ds((base + g * _G) * _FIELDS, _G * _FIELDS)],
                        idx[buf])
        pltpu.make_async_copy(v_hbm.at[idx[buf]], rows[buf], gsem[buf]).start()

    def wait_gather(buf):
        pltpu.make_async_copy(v_hbm.at[idx[buf]], rows[buf], gsem[buf]).wait()

    def start_scatter(g, buf):
        pltpu.make_async_copy(
            outv[buf],
            out_hbm.at[pl.ds((base + g * _G) * _OROW, _G * _OROW)],
            osem[buf]).start()

    def wait_scatter(g, buf):
        pltpu.make_async_copy(
            outv[buf],
            out_hbm.at[pl.ds((base + g * _G) * _OROW, _G * _OROW)],
            osem[buf]).wait()

    start_gather(0, 0)

    def outer(gg, carry):
        for b in (0, 1):
            g = gg * 2 + b

            @pl.when(g + 1 < ng)
            def _():
                start_gather(g + 1, (b + 1) % 2)

            wait_gather(b)

            @pl.when(g >= 2)
            def _():
                wait_scatter(g - 2, b)

            def inner(gb, c):
                _pair_products(rows[b], outv[b], gb)
                return c

            lax.fori_loop(0, _G, inner, 0)
            start_scatter(g, b)
        return carry

    lax.fori_loop(0, ng // 2, outer, 0)
    wait_scatter(ng - 2, 0)
    wait_scatter(ng - 1, 1)


def kernel(X, v):
    B, F = X.shape
    Vn, F2, D = v.shape
    assert F == _FIELDS and F2 == _FIELDS and D == _EMBED
    assert B % (_NW * _G) == 0
    nb = B // _NW
    ng = nb // _G
    assert ng % 2 == 0

    x_flat = X.reshape(B * F).astype(jnp.int32)
    v_flat = v.reshape(Vn, F * D)

    mesh = plsc.VectorSubcoreMesh(core_axis_name="c", subcore_axis_name="s")
    f32 = jnp.float32
    run = pl.kernel(
        functools.partial(_sc_body, nb, ng),
        mesh=mesh,
        compiler_params=pltpu.CompilerParams(use_tc_tiling_on_sc=False),
        out_type=jax.ShapeDtypeStruct((B * _OROW,), f32),
        scratch_types=[
            pltpu.VMEM((_G * _FIELDS,), jnp.int32),
            pltpu.VMEM((_G * _FIELDS,), jnp.int32),
            pltpu.VMEM((_G * _FIELDS, _ROW), f32),
            pltpu.VMEM((_G * _FIELDS, _ROW), f32),
            pltpu.VMEM((_G * _OROW,), f32),
            pltpu.VMEM((_G * _OROW,), f32),
            pltpu.SemaphoreType.DMA,
            pltpu.SemaphoreType.DMA,
            pltpu.SemaphoreType.DMA,
            pltpu.SemaphoreType.DMA,
        ],
    )
    out = run(x_flat, v_flat)
    return out.reshape(B, _NPAIRS, D)
